# Initial kernel scaffold; baseline (speedup 1.0000x reference)
import jax
import jax.numpy as jnp
from jax.experimental import pallas as pl


def kernel(x, params):
    raise NotImplementedError("write your pallas kernel here")


# speedup vs baseline: 6.3328x; 6.3328x over previous
import functools

import jax
import jax.numpy as jnp
from jax import lax
from jax.experimental import pallas as pl
from jax.experimental.pallas import tpu as pltpu
from jax.experimental.pallas import tpu_sc as plsc

KNN = 20
EPS = 1e-5
NEG = -3.0e38


def _matT(a, b):
    return lax.dot_general(a, b, (((1,), (1,)), ((), ())),
                           preferred_element_type=jnp.float32)


def _lrelu(x):
    return jnp.where(x >= 0, x, 0.2 * x)


def _stage_tc_body(xblk_ref, xall_ref, wa_ref, wd_ref, idx_ref, u_ref, v_ref):
    b = pl.program_id(0)
    xb = xblk_ref[0]
    xa = xall_ref[0]
    blk = xb.shape[0]
    n = xa.shape[0]

    g = _matT(xb, xa)
    xxb = jnp.sum(xb * xb, axis=1, keepdims=True)
    xxa = jnp.sum(xa * xa, axis=1)
    pd = (2.0 * g - xxb) - xxa[None, :]

    ids = lax.broadcasted_iota(jnp.int32, (blk, n), 1)
    kids = lax.broadcasted_iota(jnp.int32, (blk, KNN), 1)
    idx = jnp.zeros((blk, KNN), jnp.int32)
    for k in range(KNN):
        rm = jnp.max(pd, axis=1, keepdims=True)
        eq = pd == rm
        am = jnp.min(jnp.where(eq, ids, n), axis=1, keepdims=True)
        idx = jnp.where(kids == k, am, idx)
        pd = jnp.where(ids == am, NEG, pd)

    idx_ref[0] = idx + b * n
    u_ref[0] = _matT(xb, wa_ref[...])
    v_ref[0] = _matT(xb, wd_ref[...])


def _stage_tc(xt, wa, wd, blk=256):
    B, N, C = xt.shape
    O = wa.shape[0]
    grid = (B, N // blk)
    return pl.pallas_call(
        _stage_tc_body,
        grid=grid,
        in_specs=[
            pl.BlockSpec((1, blk, C), lambda b, i: (b, i, 0)),
            pl.BlockSpec((1, N, C), lambda b, i: (b, 0, 0)),
            pl.BlockSpec((O, C), lambda b, i: (0, 0)),
            pl.BlockSpec((O, C), lambda b, i: (0, 0)),
        ],
        out_specs=[
            pl.BlockSpec((1, blk, KNN), lambda b, i: (b, i, 0)),
            pl.BlockSpec((1, blk, O), lambda b, i: (b, i, 0)),
            pl.BlockSpec((1, blk, O), lambda b, i: (b, i, 0)),
        ],
        out_shape=[
            jax.ShapeDtypeStruct((B, N, KNN), jnp.int32),
            jax.ShapeDtypeStruct((B, N, O), jnp.float32),
            jax.ShapeDtypeStruct((B, N, O), jnp.float32),
        ],
    )(xt, xt, wa, wd)


def _gather_sc(u, v, idxf):
    BN, O = u.shape
    info = plsc.get_sparse_core_info()
    NC, NS = info.num_cores, info.num_subcores
    NW = NC * NS
    PW = BN // NW
    P = 4
    R = P * KNN
    ITERS = PW // P
    mesh = plsc.VectorSubcoreMesh(core_axis_name="c", subcore_axis_name="s")

    @functools.partial(
        pl.kernel,
        out_type=[
            jax.ShapeDtypeStruct((BN, O), jnp.float32),
            jax.ShapeDtypeStruct((NW, O), jnp.float32),
            jax.ShapeDtypeStruct((NW, O), jnp.float32),
        ],
        mesh=mesh,
        scratch_types=[
            pltpu.VMEM((R,), jnp.int32),
            pltpu.VMEM((R, O), jnp.float32),
            pltpu.VMEM((P, O), jnp.float32),
            pltpu.VMEM((P, O), jnp.float32),
            pltpu.VMEM((O,), jnp.float32),
            pltpu.VMEM((O,), jnp.float32),
            pltpu.SemaphoreType.DMA,
        ],
        compiler_params=pltpu.CompilerParams(use_tc_tiling_on_sc=False),
    )
    def body(u_hbm, v_hbm, idx_hbm, out_hbm, s1_hbm, s2_hbm,
             idx_v, rows_v, vv_v, out_v, a1_v, a2_v, sem):
        wid = lax.axis_index("s") * NC + lax.axis_index("c")
        pbase0 = wid * PW

        def zero_body(c, carry):
            sl = pl.ds(c * 16, 16)
            a1_v[sl] = jnp.zeros((16,), jnp.float32)
            a2_v[sl] = jnp.zeros((16,), jnp.float32)
            return carry

        lax.fori_loop(0, O // 16, zero_body, 0)

        def it_body(it, carry):
            pbase = pbase0 + it * P
            pltpu.sync_copy(idx_hbm.at[pl.ds(pbase * KNN, R)], idx_v)
            pltpu.async_copy(u_hbm.at[idx_v], rows_v, sem).wait()
            pltpu.sync_copy(v_hbm.at[pl.ds(pbase, P)], vv_v)

            def ch_body(c, inner):
                sl = pl.ds(c * 16, 16)
                for p in range(P):
                    r = rows_v[p * KNN, sl]
                    m = r
                    s = r
                    q = r * r
                    for k in range(1, KNN):
                        r = rows_v[p * KNN + k, sl]
                        m = jnp.maximum(m, r)
                        s = s + r
                        q = q + r * r
                    vv = vv_v[p, sl]
                    out_v[p, sl] = m + vv
                    a1_v[sl] = a1_v[sl] + s + float(KNN) * vv
                    a2_v[sl] = a2_v[sl] + q + 2.0 * vv * s + float(KNN) * vv * vv
                return inner

            lax.fori_loop(0, O // 16, ch_body, 0)
            pltpu.sync_copy(out_v, out_hbm.at[pl.ds(pbase, P)])
            return carry

        lax.fori_loop(0, ITERS, it_body, 0)
        pltpu.sync_copy(a1_v, s1_hbm.at[wid])
        pltpu.sync_copy(a2_v, s2_hbm.at[wid])

    return body(u, v, idxf)


def _act_tc_body(x_ref, sc_ref, sh_ref, o_ref):
    y = x_ref[...] * sc_ref[...] + sh_ref[...]
    o_ref[...] = _lrelu(y)


def _act_tc(x, scale, shift, blk=1024):
    BN, O = x.shape
    return pl.pallas_call(
        _act_tc_body,
        grid=(BN // blk,),
        in_specs=[
            pl.BlockSpec((blk, O), lambda i: (i, 0)),
            pl.BlockSpec((1, O), lambda i: (0, 0)),
            pl.BlockSpec((1, O), lambda i: (0, 0)),
        ],
        out_specs=pl.BlockSpec((blk, O), lambda i: (i, 0)),
        out_shape=jax.ShapeDtypeStruct((BN, O), jnp.float32),
    )(x, scale.reshape(1, O), shift.reshape(1, O))


def _conv5_p1_body(x1_ref, x2_ref, x3_ref, x4_ref, w1_ref, w2_ref, w3_ref,
                   w4_ref, mx_ref, s1_ref, s2_ref):
    i = pl.program_id(1)
    h = (_matT(x1_ref[0], w1_ref[...]) + _matT(x2_ref[0], w2_ref[...])
         + _matT(x3_ref[0], w3_ref[...]) + _matT(x4_ref[0], w4_ref[...]))
    hm = jnp.max(h, axis=0, keepdims=True)
    hs = jnp.sum(h, axis=0, keepdims=True)
    hq = jnp.sum(h * h, axis=0, keepdims=True)

    @pl.when(i == 0)
    def _():
        mx_ref[0] = hm
        s1_ref[0] = hs
        s2_ref[0] = hq

    @pl.when(i > 0)
    def _():
        mx_ref[0] = jnp.maximum(mx_ref[0], hm)
        s1_ref[0] = s1_ref[0] + hs
        s2_ref[0] = s2_ref[0] + hq


def _conv5_p2_body(x1_ref, x2_ref, x3_ref, x4_ref, w1_ref, w2_ref, w3_ref,
                   w4_ref, sc_ref, sh_ref, sm_ref):
    i = pl.program_id(1)
    h = (_matT(x1_ref[0], w1_ref[...]) + _matT(x2_ref[0], w2_ref[...])
         + _matT(x3_ref[0], w3_ref[...]) + _matT(x4_ref[0], w4_ref[...]))
    y = _lrelu(h * sc_ref[...] + sh_ref[...])
    ys = jnp.sum(y, axis=0, keepdims=True)

    @pl.when(i == 0)
    def _():
        sm_ref[0] = ys

    @pl.when(i > 0)
    def _():
        sm_ref[0] = sm_ref[0] + ys


def _conv5(acts, wparts, scale=None, shift=None, blk=512):
    B, N, _ = acts[0].shape
    O5 = wparts[0].shape[0]
    grid = (B, N // blk)
    in_specs = [pl.BlockSpec((1, blk, a.shape[2]), lambda b, i: (b, i, 0))
                for a in acts]
    in_specs += [pl.BlockSpec(w.shape, lambda b, i: (0, 0)) for w in wparts]
    out_spec = pl.BlockSpec((1, 1, O5), lambda b, i: (b, 0, 0))
    out_shape = jax.ShapeDtypeStruct((B, 1, O5), jnp.float32)
    if scale is None:
        return pl.pallas_call(
            _conv5_p1_body, grid=grid, in_specs=in_specs,
            out_specs=[out_spec] * 3, out_shape=[out_shape] * 3,
        )(*acts, *wparts)
    in_specs += [pl.BlockSpec((1, O5), lambda b, i: (0, 0))] * 2
    return pl.pallas_call(
        _conv5_p2_body, grid=grid, in_specs=in_specs,
        out_specs=out_spec, out_shape=out_shape,
    )(*acts, *wparts, scale.reshape(1, O5), shift.reshape(1, O5))


def _head_body(mx_ref, sm_ref, sc_ref, sh_ref, l1_ref, g6_ref, b6_ref,
               l2_ref, l2b_ref, g7_ref, b7_ref, l3_ref, l3b_ref, o_ref,
               n_points):
    p1 = _lrelu(mx_ref[...] * sc_ref[...] + sh_ref[...])
    p2 = sm_ref[...] * (1.0 / n_points)
    z = jnp.concatenate([p1, p2], axis=1)

    def bn0(a, g, b):
        m = jnp.mean(a, axis=0, keepdims=True)
        v = jnp.mean(a * a, axis=0, keepdims=True) - m * m
        return (a - m) * lax.rsqrt(v + EPS) * g + b

    z = _lrelu(bn0(_matT(z, l1_ref[...]), g6_ref[...], b6_ref[...]))
    z = _lrelu(bn0(_matT(z, l2_ref[...]) + l2b_ref[...], g7_ref[...],
                   b7_ref[...]))
    o_ref[...] = _matT(z, l3_ref[...]) + l3b_ref[...]


def _head(mx, sm, scale, shift, p, n_points):
    B, O5 = mx.shape
    return pl.pallas_call(
        functools.partial(_head_body, n_points=float(n_points)),
        out_shape=jax.ShapeDtypeStruct((B, 40), jnp.float32),
    )(mx, sm, scale.reshape(1, O5), shift.reshape(1, O5),
      p['L1'], p['g6'].reshape(1, -1), p['b6'].reshape(1, -1),
      p['L2'], p['L2b'].reshape(1, -1), p['g7'].reshape(1, -1),
      p['b7'].reshape(1, -1), p['L3'], p['L3b'].reshape(1, -1))


def _edge_stage(xt, w, g, b):
    B, N, C = xt.shape
    O = w.shape[0]
    wa = w[:, :C]
    wd = w[:, C:] - wa
    if C < 8:
        pad = 8 - C
        xt = jnp.pad(xt, ((0, 0), (0, 0), (0, pad)))
        wa = jnp.pad(wa, ((0, 0), (0, pad)))
        wd = jnp.pad(wd, ((0, 0), (0, pad)))
    idx, u, v = _stage_tc(xt, wa, wd)
    BN = B * N
    ypre, s1p, s2p = _gather_sc(u.reshape(BN, O), v.reshape(BN, O),
                                idx.reshape(BN * KNN))
    cnt = BN * KNN
    s1 = jnp.sum(s1p, axis=0)
    s2 = jnp.sum(s2p, axis=0)
    mean = s1 / cnt
    var = s2 / cnt - mean * mean
    scale = g / jnp.sqrt(var + EPS)
    shift = b - mean * scale
    y = _act_tc(ypre, scale, shift)
    return y.reshape(B, N, O)


def kernel(x, params):
    p = params
    B, _, N = x.shape
    xt = jnp.transpose(x, (0, 2, 1))
    x1 = _edge_stage(xt, p['W1'], p['g1'], p['b1'])
    x2 = _edge_stage(x1, p['W2'], p['g2'], p['b2'])
    x3 = _edge_stage(x2, p['W3'], p['g3'], p['b3'])
    x4 = _edge_stage(x3, p['W4'], p['g4'], p['b4'])

    acts = [x1, x2, x3, x4]
    w5 = p['W5']
    splits = [0, 64, 128, 256, 512]
    wparts = [w5[:, splits[i]:splits[i + 1]] for i in range(4)]
    mx, s1p, s2p = _conv5(acts, wparts)
    mx = mx.reshape(B, -1)
    cnt = B * N
    s1 = jnp.sum(s1p, axis=(0, 1))
    s2 = jnp.sum(s2p, axis=(0, 1))
    mean = s1 / cnt
    var = s2 / cnt - mean * mean
    scale = p['g5'] / jnp.sqrt(var + EPS)
    shift = p['b5'] - mean * scale
    sm = _conv5(acts, wparts, scale, shift).reshape(B, -1)
    return _head(mx, sm, scale, shift, p, N)
